# Initial kernel scaffold; baseline (speedup 1.0000x reference)
import jax
import jax.numpy as jnp
from jax.experimental import pallas as pl


def kernel(x, edge_index, Wq, bq, Wk, bk, Wv, bv, tw, Wo, bo):
    raise NotImplementedError("write your pallas kernel here")


# speedup vs baseline: 14.2061x; 14.2061x over previous
import functools

import jax
import jax.numpy as jnp
from jax import lax
from jax.experimental import pallas as pl
from jax.experimental.pallas import tpu as pltpu
from jax.experimental.pallas import tpu_sc as plsc

_N = 10000
_E = 320000
_HID = 128
_H = 8
_DH = 16
_SCALE = 1.0 / (_DH ** 0.5)

_NC = 2
_NS = 16
_NW = _NC * _NS
_EW = _E // _NW
_C1 = 400
_NCH1 = _EW // _C1
_C2 = 80
_SUB = _C1 // _C2
_ACC_D = 144

_SC_PARAMS = pltpu.CompilerParams(needs_layout_passes=False,
                                  use_tc_tiling_on_sc=False)


def _qkv_body(x_ref, wq_ref, bq_ref, wk_ref, bk_ref, wv_ref, bv_ref,
              q_ref, k_ref, v_ref):
    xb = x_ref[...]
    dn = (((1,), (1,)), ((), ()))
    q_ref[...] = lax.dot_general(xb, wq_ref[...], dn,
                                 preferred_element_type=jnp.float32) + bq_ref[...]
    k_ref[...] = lax.dot_general(xb, wk_ref[...], dn,
                                 preferred_element_type=jnp.float32) + bk_ref[...]
    v_ref[...] = lax.dot_general(xb, wv_ref[...], dn,
                                 preferred_element_type=jnp.float32) + bv_ref[...]


def _qkv(x, Wq, bq, Wk, bk, Wv, bv):
    br = 1000
    wspec = pl.BlockSpec((_HID, _HID), lambda i: (0, 0))
    bspec = pl.BlockSpec((1, _HID), lambda i: (0, 0))
    rspec = pl.BlockSpec((br, _HID), lambda i: (i, 0))
    return pl.pallas_call(
        _qkv_body,
        grid=(_N // br,),
        in_specs=[rspec, wspec, bspec, wspec, bspec, wspec, bspec],
        out_specs=[rspec, rspec, rspec],
        out_shape=[jax.ShapeDtypeStruct((_N, _HID), jnp.float32)] * 3,
    )(x, Wq, bq.reshape(1, _HID), Wk, bk.reshape(1, _HID), Wv, bv.reshape(1, _HID))


def _sc_scores(q, k, row, col):
    mesh = plsc.VectorSubcoreMesh(core_axis_name="c", subcore_axis_name="s")

    @functools.partial(
        pl.kernel,
        mesh=mesh,
        compiler_params=_SC_PARAMS,
        out_type=jax.ShapeDtypeStruct((_NW, _NCH1, _H, _C1), jnp.float32),
        scratch_types=[
            pltpu.VMEM((_C1,), jnp.int32),
            pltpu.VMEM((_C1,), jnp.int32),
            pltpu.VMEM((_C1, _HID), jnp.float32),
            pltpu.VMEM((_C1, _HID), jnp.float32),
            pltpu.VMEM((_H, _C1), jnp.float32),
            pltpu.SemaphoreType.DMA,
        ],
    )
    def kfn(q_hbm, k_hbm, row_hbm, col_hbm, s_hbm, colv, rowv, qv, kv, sv, sem):
        wid = lax.axis_index("s") * _NC + lax.axis_index("c")

        def chunk_body(ci, carry):
            base = wid * _EW + ci * _C1
            pltpu.sync_copy(col_hbm.at[pl.ds(base, _C1)], colv)
            pltpu.sync_copy(row_hbm.at[pl.ds(base, _C1)], rowv)
            pltpu.async_copy(q_hbm.at[colv], qv, sem).wait()
            pltpu.async_copy(k_hbm.at[rowv], kv, sem).wait()

            def eb_body(eb, c2):
                ei = eb * 16 + lax.iota(jnp.int32, 16)
                for h in range(_H):
                    acc = jnp.zeros((16,), jnp.float32)
                    for d in range(_DH):
                        cidx = jnp.full((16,), h * _DH + d, jnp.int32)
                        qg = plsc.load_gather(qv, [ei, cidx])
                        kg = plsc.load_gather(kv, [ei, cidx])
                        acc = acc + qg * kg
                    sv[h, pl.ds(eb * 16, 16)] = acc * _SCALE
                return c2

            lax.fori_loop(0, _C1 // 16, eb_body, 0)
            pltpu.sync_copy(sv, s_hbm.at[wid, ci])
            return carry

        lax.fori_loop(0, _NCH1, chunk_body, 0)

    return kfn(q, k, row, col)


def _softmax_c_body(s_ref, o_ref):
    s = s_ref[...]
    m = jnp.max(jnp.max(s, axis=2), axis=0)
    e = jnp.exp(s - m[None, :, None])
    z = jnp.sum(jnp.sum(e, axis=2), axis=0)
    c = m + jnp.log(z)
    o_ref[...] = jnp.broadcast_to(c[:, None], (_H, 128))


def _softmax_c(scores):
    s3 = scores.reshape(_NW * _NCH1, _H, _C1)
    return pl.pallas_call(
        _softmax_c_body,
        out_shape=jax.ShapeDtypeStruct((_H, 128), jnp.float32),
    )(s3)


def _sc_scatter(v, scores, row2, col2, c8):
    mesh = plsc.VectorSubcoreMesh(core_axis_name="c", subcore_axis_name="s")

    @functools.partial(
        pl.kernel,
        mesh=mesh,
        compiler_params=_SC_PARAMS,
        out_type=jax.ShapeDtypeStruct((_NC, _N, _ACC_D), jnp.float32),
        scratch_types=[
            pltpu.VMEM((_SUB, _C2), jnp.int32),
            pltpu.VMEM((_SUB, _C2), jnp.int32),
            pltpu.VMEM((_C2, _HID), jnp.float32),
            pltpu.VMEM((_C2, _ACC_D), jnp.float32),
            pltpu.VMEM((_H, _C1), jnp.float32),
            pltpu.VMEM((_H, 16), jnp.float32),
            pltpu.VMEM_SHARED((_N, _ACC_D), jnp.float32),
            pltpu.SemaphoreType.DMA,
        ],
    )
    def kfn(v_hbm, s_hbm, row_hbm, col_hbm, c_hbm, out_hbm,
            colv, rowv, vv, valsv, av, cv, acc_sh, sem):
        cid = lax.axis_index("c")
        sid = lax.axis_index("s")
        wid = sid * _NC + cid

        pltpu.sync_copy(c_hbm, cv)

        def zbuf_body(e, carry):
            for j in range(_ACC_D // 16):
                valsv[e, pl.ds(j * 16, 16)] = jnp.zeros((16,), jnp.float32)
            return carry

        lax.fori_loop(0, _C2, zbuf_body, 0)

        r0 = sid * (_N // _NS)
        nfull = (_N // _NS) // _C2
        rem = (_N // _NS) - nfull * _C2

        def zacc_body(i, carry):
            pltpu.sync_copy(valsv, acc_sh.at[pl.ds(r0 + i * _C2, _C2)])
            return carry

        lax.fori_loop(0, nfull, zacc_body, 0)
        pltpu.sync_copy(valsv.at[pl.ds(0, rem)],
                        acc_sh.at[pl.ds(r0 + nfull * _C2, rem)])
        plsc.subcore_barrier()

        def macro_body(mc, carry):
            mrow = wid * (_EW // _C2) + mc * _SUB
            pltpu.sync_copy(col_hbm.at[pl.ds(mrow, _SUB)], colv)
            pltpu.sync_copy(row_hbm.at[pl.ds(mrow, _SUB)], rowv)
            pltpu.sync_copy(s_hbm.at[wid, mc], av)

            def exp_body(eb, c2):
                sl = pl.ds(eb * 16, 16)
                for h in range(_H):
                    av[h, sl] = jnp.exp(av[h, sl] - cv[h, pl.ds(0, 16)])
                return c2

            lax.fori_loop(0, _C1 // 16, exp_body, 0)

            def sub_body(sub, c3):
                pltpu.async_copy(v_hbm.at[rowv.at[sub]], vv, sem).wait()

                def eb_body(eb, c4):
                    ei = eb * 16 + lax.iota(jnp.int32, 16)
                    off = sub * _C2 + eb * 16
                    asum = jnp.zeros((16,), jnp.float32)
                    for h in range(_H):
                        a = av[h, pl.ds(off, 16)]
                        asum = asum + a
                        for d in range(_DH):
                            cidx = jnp.full((16,), h * _DH + d, jnp.int32)
                            vg = plsc.load_gather(vv, [ei, cidx])
                            plsc.store_scatter(valsv, [ei, cidx], vg * a)
                    plsc.store_scatter(
                        valsv, [ei, jnp.full((16,), _HID, jnp.int32)],
                        asum * (1.0 / _H))
                    return c4

                lax.fori_loop(0, _C2 // 16, eb_body, 0)
                pltpu.sync_copy(valsv, acc_sh.at[colv.at[sub]], add=True)
                return c3

            lax.fori_loop(0, _SUB, sub_body, 0)
            return carry

        lax.fori_loop(0, _NCH1, macro_body, 0)
        plsc.subcore_barrier()
        pltpu.sync_copy(acc_sh.at[pl.ds(r0, _N // _NS)],
                        out_hbm.at[cid, pl.ds(r0, _N // _NS)])

    return kfn(v, scores, row2, col2, c8)


def _final_body(a0_ref, a1_ref, wo_ref, bo_ref, o_ref):
    a = a0_ref[...] + a1_ref[...]
    att = a[:, :_HID]
    tws = jnp.maximum(a[:, _HID:_HID + 1], 1e-8)
    agg = att / tws
    o_ref[...] = lax.dot_general(agg, wo_ref[...], (((1,), (1,)), ((), ())),
                                 preferred_element_type=jnp.float32) + bo_ref[...]


def _final(acc0, acc1, Wo, bo2):
    br = 1000
    aspec = pl.BlockSpec((br, _ACC_D), lambda i: (i, 0))
    return pl.pallas_call(
        _final_body,
        grid=(_N // br,),
        in_specs=[aspec, aspec,
                  pl.BlockSpec((_HID, _HID), lambda i: (0, 0)),
                  pl.BlockSpec((1, _HID), lambda i: (0, 0))],
        out_specs=pl.BlockSpec((br, _HID), lambda i: (i, 0)),
        out_shape=jax.ShapeDtypeStruct((_N, _HID), jnp.float32),
    )(acc0, acc1, Wo, bo2)


def kernel(x, edge_index, Wq, bq, Wk, bk, Wv, bv, tw, Wo, bo):
    row = edge_index[0].astype(jnp.int32)
    col = edge_index[1].astype(jnp.int32)
    q, k, v = _qkv(x, Wq, bq, Wk, bk, Wv, bv)
    scores = _sc_scores(q, k, row, col)
    cb = _softmax_c(scores)
    acc = _sc_scatter(v, scores,
                      row.reshape(_E // _C2, _C2), col.reshape(_E // _C2, _C2),
                      cb[:, :16])
    return _final(acc[0], acc[1], Wo, bo.reshape(1, _HID))


# speedup vs baseline: 36.4900x; 2.5686x over previous
import functools

import jax
import jax.numpy as jnp
from jax import lax
from jax.experimental import pallas as pl
from jax.experimental.pallas import tpu as pltpu
from jax.experimental.pallas import tpu_sc as plsc

_N = 10000
_E = 320000
_HID = 128
_H = 8
_DH = 16
_SCALE = 1.0 / (_DH ** 0.5)

_NC = 2
_NS = 16
_NW = _NC * _NS
_EW = _E // _NW
_C1 = 400
_NCH1 = _EW // _C1
_C2 = 80
_SUB = _C1 // _C2
_ACC_D = 144

_SC_PARAMS = pltpu.CompilerParams(needs_layout_passes=False,
                                  use_tc_tiling_on_sc=False)


def _qkv_body(x_ref, wq_ref, bq_ref, wk_ref, bk_ref, wv_ref, bv_ref,
              q_ref, k_ref, v_ref):
    xb = x_ref[...]
    dn = (((1,), (1,)), ((), ()))
    q_ref[...] = lax.dot_general(xb, wq_ref[...], dn,
                                 preferred_element_type=jnp.float32) + bq_ref[...]
    k_ref[...] = lax.dot_general(xb, wk_ref[...], dn,
                                 preferred_element_type=jnp.float32) + bk_ref[...]
    v_ref[...] = lax.dot_general(xb, wv_ref[...], dn,
                                 preferred_element_type=jnp.float32) + bv_ref[...]


def _qkv(x, Wq, bq, Wk, bk, Wv, bv):
    br = 1000
    wspec = pl.BlockSpec((_HID, _HID), lambda i: (0, 0))
    bspec = pl.BlockSpec((1, _HID), lambda i: (0, 0))
    rspec = pl.BlockSpec((br, _HID), lambda i: (i, 0))
    return pl.pallas_call(
        _qkv_body,
        grid=(_N // br,),
        in_specs=[rspec, wspec, bspec, wspec, bspec, wspec, bspec],
        out_specs=[rspec, rspec, rspec],
        out_shape=[jax.ShapeDtypeStruct((_N, _HID), jnp.float32)] * 3,
    )(x, Wq, bq.reshape(1, _HID), Wk, bk.reshape(1, _HID), Wv, bv.reshape(1, _HID))


def _sc_scores(q, k, row, col):
    mesh = plsc.VectorSubcoreMesh(core_axis_name="c", subcore_axis_name="s")

    @functools.partial(
        pl.kernel,
        mesh=mesh,
        compiler_params=_SC_PARAMS,
        out_type=jax.ShapeDtypeStruct((_NW, _NCH1, _H, _C1), jnp.float32),
        scratch_types=[
            pltpu.VMEM((_C1,), jnp.int32),
            pltpu.VMEM((_C1,), jnp.int32),
            pltpu.VMEM((_C1, _HID), jnp.float32),
            pltpu.VMEM((_C1, _HID), jnp.float32),
            pltpu.VMEM((_H, _C1), jnp.float32),
            pltpu.SemaphoreType.DMA,
        ],
    )
    def kfn(q_hbm, k_hbm, row_hbm, col_hbm, s_hbm, colv, rowv, qv, kv, sv, sem):
        wid = lax.axis_index("s") * _NC + lax.axis_index("c")

        def chunk_body(ci, carry):
            base = wid * _EW + ci * _C1
            pltpu.sync_copy(col_hbm.at[pl.ds(base, _C1)], colv)
            pltpu.sync_copy(row_hbm.at[pl.ds(base, _C1)], rowv)
            pltpu.async_copy(q_hbm.at[colv], qv, sem).wait()
            pltpu.async_copy(k_hbm.at[rowv], kv, sem).wait()

            def eb_body(eb, c2):
                lane = lax.iota(jnp.int32, 16)
                ei = eb * 16 + lane
                for h in range(_H):
                    acc = jnp.zeros((16,), jnp.float32)
                    for d in range(_DH):
                        cidx = h * _DH + ((d + lane) & (_DH - 1))
                        qg = plsc.load_gather(qv, [ei, cidx])
                        kg = plsc.load_gather(kv, [ei, cidx])
                        acc = acc + qg * kg
                    sv[h, pl.ds(eb * 16, 16)] = acc * _SCALE
                return c2

            lax.fori_loop(0, _C1 // 16, eb_body, 0)
            pltpu.sync_copy(sv, s_hbm.at[wid, ci])
            return carry

        lax.fori_loop(0, _NCH1, chunk_body, 0)

    return kfn(q, k, row, col)


def _softmax_c_body(s_ref, o_ref):
    s = s_ref[...]
    m = jnp.max(jnp.max(s, axis=2), axis=0)
    e = jnp.exp(s - m[None, :, None])
    z = jnp.sum(jnp.sum(e, axis=2), axis=0)
    c = m + jnp.log(z)
    o_ref[...] = jnp.broadcast_to(c[:, None], (_H, 128))


def _softmax_c(scores):
    s3 = scores.reshape(_NW * _NCH1, _H, _C1)
    return pl.pallas_call(
        _softmax_c_body,
        out_shape=jax.ShapeDtypeStruct((_H, 128), jnp.float32),
    )(s3)


def _sc_scatter(v, scores, row2, col2, c8):
    mesh = plsc.VectorSubcoreMesh(core_axis_name="c", subcore_axis_name="s")

    @functools.partial(
        pl.kernel,
        mesh=mesh,
        compiler_params=_SC_PARAMS,
        out_type=jax.ShapeDtypeStruct((_NC, _N, _ACC_D), jnp.float32),
        scratch_types=[
            pltpu.VMEM((_SUB, _C2), jnp.int32),
            pltpu.VMEM((_SUB, _C2), jnp.int32),
            pltpu.VMEM((_C2, _HID), jnp.float32),
            pltpu.VMEM((_C2, _ACC_D), jnp.float32),
            pltpu.VMEM((_H, _C1), jnp.float32),
            pltpu.VMEM((_H, 16), jnp.float32),
            pltpu.VMEM_SHARED((_N, _ACC_D), jnp.float32),
            pltpu.SemaphoreType.DMA,
        ],
    )
    def kfn(v_hbm, s_hbm, row_hbm, col_hbm, c_hbm, out_hbm,
            colv, rowv, vv, valsv, av, cv, acc_sh, sem):
        cid = lax.axis_index("c")
        sid = lax.axis_index("s")
        wid = sid * _NC + cid

        pltpu.sync_copy(c_hbm, cv)

        def zbuf_body(e, carry):
            for j in range(_ACC_D // 16):
                valsv[e, pl.ds(j * 16, 16)] = jnp.zeros((16,), jnp.float32)
            return carry

        lax.fori_loop(0, _C2, zbuf_body, 0)

        r0 = sid * (_N // _NS)
        nfull = (_N // _NS) // _C2
        rem = (_N // _NS) - nfull * _C2

        def zacc_body(i, carry):
            pltpu.sync_copy(valsv, acc_sh.at[pl.ds(r0 + i * _C2, _C2)])
            return carry

        lax.fori_loop(0, nfull, zacc_body, 0)
        pltpu.sync_copy(valsv.at[pl.ds(0, rem)],
                        acc_sh.at[pl.ds(r0 + nfull * _C2, rem)])
        plsc.subcore_barrier()

        def macro_body(mc, carry):
            mrow = wid * (_EW // _C2) + mc * _SUB
            pltpu.sync_copy(col_hbm.at[pl.ds(mrow, _SUB)], colv)
            pltpu.sync_copy(row_hbm.at[pl.ds(mrow, _SUB)], rowv)
            pltpu.sync_copy(s_hbm.at[wid, mc], av)

            def exp_body(eb, c2):
                sl = pl.ds(eb * 16, 16)
                for h in range(_H):
                    av[h, sl] = jnp.exp(av[h, sl] - cv[h, pl.ds(0, 16)])
                return c2

            lax.fori_loop(0, _C1 // 16, exp_body, 0)

            def sub_body(sub, c3):
                pltpu.async_copy(v_hbm.at[rowv.at[sub]], vv, sem).wait()

                def eb_body(eb, c4):
                    lane = lax.iota(jnp.int32, 16)
                    ei = eb * 16 + lane
                    off = sub * _C2 + eb * 16
                    asum = jnp.zeros((16,), jnp.float32)
                    for h in range(_H):
                        a = av[h, pl.ds(off, 16)]
                        asum = asum + a
                        for d in range(_DH):
                            cidx = h * _DH + ((d + lane) & (_DH - 1))
                            vg = plsc.load_gather(vv, [ei, cidx])
                            plsc.store_scatter(valsv, [ei, cidx], vg * a)
                    twv = asum * (1.0 / _H)
                    for d in range(_DH):
                        cidx = _HID + ((d + lane) & (_DH - 1))
                        plsc.store_scatter(valsv, [ei, cidx], twv)
                    return c4

                lax.fori_loop(0, _C2 // 16, eb_body, 0)
                pltpu.sync_copy(valsv, acc_sh.at[colv.at[sub]], add=True)
                return c3

            lax.fori_loop(0, _SUB, sub_body, 0)
            return carry

        lax.fori_loop(0, _NCH1, macro_body, 0)
        plsc.subcore_barrier()
        pltpu.sync_copy(acc_sh.at[pl.ds(r0, _N // _NS)],
                        out_hbm.at[cid, pl.ds(r0, _N // _NS)])

    return kfn(v, scores, row2, col2, c8)


def _final_body(a0_ref, a1_ref, wo_ref, bo_ref, o_ref):
    a = a0_ref[...] + a1_ref[...]
    att = a[:, :_HID]
    tws = jnp.maximum(a[:, _HID:_HID + 1], 1e-8)
    agg = att / tws
    o_ref[...] = lax.dot_general(agg, wo_ref[...], (((1,), (1,)), ((), ())),
                                 preferred_element_type=jnp.float32) + bo_ref[...]


def _final(acc0, acc1, Wo, bo2):
    br = 1000
    aspec = pl.BlockSpec((br, _ACC_D), lambda i: (i, 0))
    return pl.pallas_call(
        _final_body,
        grid=(_N // br,),
        in_specs=[aspec, aspec,
                  pl.BlockSpec((_HID, _HID), lambda i: (0, 0)),
                  pl.BlockSpec((1, _HID), lambda i: (0, 0))],
        out_specs=pl.BlockSpec((br, _HID), lambda i: (i, 0)),
        out_shape=jax.ShapeDtypeStruct((_N, _HID), jnp.float32),
    )(acc0, acc1, Wo, bo2)


def kernel(x, edge_index, Wq, bq, Wk, bk, Wv, bv, tw, Wo, bo):
    row = edge_index[0].astype(jnp.int32)
    col = edge_index[1].astype(jnp.int32)
    q, k, v = _qkv(x, Wq, bq, Wk, bk, Wv, bv)
    scores = _sc_scores(q, k, row, col)
    cb = _softmax_c(scores)
    acc = _sc_scatter(v, scores,
                      row.reshape(_E // _C2, _C2), col.reshape(_E // _C2, _C2),
                      cb[:, :16])
    return _final(acc[0], acc[1], Wo, bo.reshape(1, _HID))


# speedup vs baseline: 38.8045x; 1.0634x over previous
import functools

import jax
import jax.numpy as jnp
from jax import lax
from jax.experimental import pallas as pl
from jax.experimental.pallas import tpu as pltpu
from jax.experimental.pallas import tpu_sc as plsc

_N = 10000
_E = 320000
_HID = 128
_H = 8
_DH = 16
_SCALE = 1.0 / (_DH ** 0.5)

_NC = 2
_NS = 16
_NW = _NC * _NS
_EW = _E // _NW
_C1 = 400
_NCH1 = _EW // _C1
_C2 = 80
_SUB = _C1 // _C2
_ACC_D = 144

_SC_PARAMS = pltpu.CompilerParams(needs_layout_passes=False,
                                  use_tc_tiling_on_sc=False)


def _qkv_body(x_ref, wq_ref, bq_ref, wk_ref, bk_ref, wv_ref, bv_ref,
              q_ref, k_ref, v_ref):
    xb = x_ref[...]
    dn = (((1,), (1,)), ((), ()))
    q_ref[...] = lax.dot_general(xb, wq_ref[...], dn,
                                 preferred_element_type=jnp.float32) + bq_ref[...]
    k_ref[...] = lax.dot_general(xb, wk_ref[...], dn,
                                 preferred_element_type=jnp.float32) + bk_ref[...]
    v_ref[...] = lax.dot_general(xb, wv_ref[...], dn,
                                 preferred_element_type=jnp.float32) + bv_ref[...]


def _qkv(x, Wq, bq, Wk, bk, Wv, bv):
    br = 1000
    wspec = pl.BlockSpec((_HID, _HID), lambda i: (0, 0))
    bspec = pl.BlockSpec((1, _HID), lambda i: (0, 0))
    rspec = pl.BlockSpec((br, _HID), lambda i: (i, 0))
    return pl.pallas_call(
        _qkv_body,
        grid=(_N // br,),
        in_specs=[rspec, wspec, bspec, wspec, bspec, wspec, bspec],
        out_specs=[rspec, rspec, rspec],
        out_shape=[jax.ShapeDtypeStruct((_N, _HID), jnp.float32)] * 3,
    )(x, Wq, bq.reshape(1, _HID), Wk, bk.reshape(1, _HID), Wv, bv.reshape(1, _HID))


def _sc_scores(q, k, row, col):
    mesh = plsc.VectorSubcoreMesh(core_axis_name="c", subcore_axis_name="s")

    @functools.partial(
        pl.kernel,
        mesh=mesh,
        compiler_params=_SC_PARAMS,
        out_type=jax.ShapeDtypeStruct((_NW, _NCH1, _H, _C1), jnp.float32),
        scratch_types=[
            pltpu.VMEM((_C1,), jnp.int32),
            pltpu.VMEM((_C1,), jnp.int32),
            pltpu.VMEM((_C1, _HID), jnp.float32),
            pltpu.VMEM((_C1, _HID), jnp.float32),
            pltpu.VMEM((_H, _C1), jnp.float32),
            pltpu.SemaphoreType.DMA,
            pltpu.SemaphoreType.DMA,
        ],
    )
    def kfn(q_hbm, k_hbm, row_hbm, col_hbm, s_hbm, colv, rowv, qv, kv, sv,
            sem, sem2):
        wid = lax.axis_index("s") * _NC + lax.axis_index("c")

        def chunk_body(ci, carry):
            base = wid * _EW + ci * _C1
            cd = pltpu.async_copy(col_hbm.at[pl.ds(base, _C1)], colv, sem)
            rd = pltpu.async_copy(row_hbm.at[pl.ds(base, _C1)], rowv, sem2)
            cd.wait()
            qd = pltpu.async_copy(q_hbm.at[colv], qv, sem)
            rd.wait()
            kd = pltpu.async_copy(k_hbm.at[rowv], kv, sem2)
            qd.wait()
            kd.wait()

            def eb_body(eb, c2):
                lane = lax.iota(jnp.int32, 16)
                ei = eb * 16 + lane
                for h in range(_H):
                    acc = jnp.zeros((16,), jnp.float32)
                    for d in range(_DH):
                        cidx = h * _DH + ((d + lane) & (_DH - 1))
                        qg = plsc.load_gather(qv, [ei, cidx])
                        kg = plsc.load_gather(kv, [ei, cidx])
                        acc = acc + qg * kg
                    sv[h, pl.ds(eb * 16, 16)] = acc * _SCALE
                return c2

            lax.fori_loop(0, _C1 // 16, eb_body, 0)
            pltpu.sync_copy(sv, s_hbm.at[wid, ci])
            return carry

        lax.fori_loop(0, _NCH1, chunk_body, 0)

    return kfn(q, k, row, col)


def _softmax_c_body(s_ref, o_ref):
    s = s_ref[...]
    m = jnp.max(jnp.max(s, axis=2), axis=0)
    e = jnp.exp(s - m[None, :, None])
    z = jnp.sum(jnp.sum(e, axis=2), axis=0)
    c = m + jnp.log(z)
    o_ref[...] = jnp.broadcast_to(c[:, None], (_H, 128))


def _softmax_c(scores):
    s3 = scores.reshape(_NW * _NCH1, _H, _C1)
    return pl.pallas_call(
        _softmax_c_body,
        out_shape=jax.ShapeDtypeStruct((_H, 128), jnp.float32),
    )(s3)


def _sc_scatter(v, scores, row2, col2, c8):
    mesh = plsc.VectorSubcoreMesh(core_axis_name="c", subcore_axis_name="s")

    @functools.partial(
        pl.kernel,
        mesh=mesh,
        compiler_params=_SC_PARAMS,
        out_type=jax.ShapeDtypeStruct((_NC, _N, _ACC_D), jnp.float32),
        scratch_types=[
            pltpu.VMEM((_SUB, _C2), jnp.int32),
            pltpu.VMEM((_SUB, _C2), jnp.int32),
            pltpu.VMEM((_C2, _HID), jnp.float32),
            pltpu.VMEM((_C2, _ACC_D), jnp.float32),
            pltpu.VMEM((_H, _C1), jnp.float32),
            pltpu.VMEM((_H, 16), jnp.float32),
            pltpu.VMEM_SHARED((_N, _ACC_D), jnp.float32),
            pltpu.SemaphoreType.DMA,
            pltpu.SemaphoreType.DMA,
        ],
    )
    def kfn(v_hbm, s_hbm, row_hbm, col_hbm, c_hbm, out_hbm,
            colv, rowv, vv, valsv, av, cv, acc_sh, sem, sem_s):
        cid = lax.axis_index("c")
        sid = lax.axis_index("s")
        wid = sid * _NC + cid

        pltpu.sync_copy(c_hbm, cv)

        def zbuf_body(e, carry):
            for j in range(_ACC_D // 16):
                valsv[e, pl.ds(j * 16, 16)] = jnp.zeros((16,), jnp.float32)
            return carry

        lax.fori_loop(0, _C2, zbuf_body, 0)

        r0 = sid * (_N // _NS)
        nfull = (_N // _NS) // _C2
        rem = (_N // _NS) - nfull * _C2

        def zacc_body(i, carry):
            pltpu.sync_copy(valsv, acc_sh.at[pl.ds(r0 + i * _C2, _C2)])
            return carry

        lax.fori_loop(0, nfull, zacc_body, 0)
        pltpu.sync_copy(valsv.at[pl.ds(0, rem)],
                        acc_sh.at[pl.ds(r0 + nfull * _C2, rem)])
        plsc.subcore_barrier()

        def macro_body(mc, carry):
            mrow = wid * (_EW // _C2) + mc * _SUB
            pltpu.sync_copy(col_hbm.at[pl.ds(mrow, _SUB)], colv)
            pltpu.sync_copy(row_hbm.at[pl.ds(mrow, _SUB)], rowv)
            pltpu.sync_copy(s_hbm.at[wid, mc], av)

            def exp_body(eb, c2):
                sl = pl.ds(eb * 16, 16)
                for h in range(_H):
                    av[h, sl] = jnp.exp(av[h, sl] - cv[h, pl.ds(0, 16)])
                return c2

            lax.fori_loop(0, _C1 // 16, exp_body, 0)

            def sub_body(sub, c3):
                gd = pltpu.async_copy(v_hbm.at[rowv.at[sub]], vv, sem)

                @pl.when(sub > 0)
                def _():
                    pltpu.make_async_copy(
                        valsv, acc_sh.at[colv.at[0]], sem_s).wait()

                gd.wait()

                def eb_body(eb, c4):
                    lane = lax.iota(jnp.int32, 16)
                    ei = eb * 16 + lane
                    off = sub * _C2 + eb * 16
                    asum = jnp.zeros((16,), jnp.float32)
                    for h in range(_H):
                        a = av[h, pl.ds(off, 16)]
                        asum = asum + a
                        for d in range(_DH):
                            cidx = h * _DH + ((d + lane) & (_DH - 1))
                            vg = plsc.load_gather(vv, [ei, cidx])
                            plsc.store_scatter(valsv, [ei, cidx], vg * a)
                    twv = asum * (1.0 / _H)
                    for d in range(_DH):
                        cidx = _HID + ((d + lane) & (_DH - 1))
                        plsc.store_scatter(valsv, [ei, cidx], twv)
                    return c4

                lax.fori_loop(0, _C2 // 16, eb_body, 0)
                pltpu.async_copy(valsv, acc_sh.at[colv.at[sub]], sem_s,
                                 add=True)
                return c3

            lax.fori_loop(0, _SUB, sub_body, 0)
            pltpu.make_async_copy(valsv, acc_sh.at[colv.at[0]], sem_s).wait()
            return carry

        lax.fori_loop(0, _NCH1, macro_body, 0)
        plsc.subcore_barrier()
        pltpu.sync_copy(acc_sh.at[pl.ds(r0, _N // _NS)],
                        out_hbm.at[cid, pl.ds(r0, _N // _NS)])

    return kfn(v, scores, row2, col2, c8)


def _final_body(a0_ref, a1_ref, wo_ref, bo_ref, o_ref):
    a = a0_ref[...] + a1_ref[...]
    att = a[:, :_HID]
    tws = jnp.maximum(a[:, _HID:_HID + 1], 1e-8)
    agg = att / tws
    o_ref[...] = lax.dot_general(agg, wo_ref[...], (((1,), (1,)), ((), ())),
                                 preferred_element_type=jnp.float32) + bo_ref[...]


def _final(acc0, acc1, Wo, bo2):
    br = 1000
    aspec = pl.BlockSpec((br, _ACC_D), lambda i: (i, 0))
    return pl.pallas_call(
        _final_body,
        grid=(_N // br,),
        in_specs=[aspec, aspec,
                  pl.BlockSpec((_HID, _HID), lambda i: (0, 0)),
                  pl.BlockSpec((1, _HID), lambda i: (0, 0))],
        out_specs=pl.BlockSpec((br, _HID), lambda i: (i, 0)),
        out_shape=jax.ShapeDtypeStruct((_N, _HID), jnp.float32),
    )(acc0, acc1, Wo, bo2)


def kernel(x, edge_index, Wq, bq, Wk, bk, Wv, bv, tw, Wo, bo):
    row = edge_index[0].astype(jnp.int32)
    col = edge_index[1].astype(jnp.int32)
    q, k, v = _qkv(x, Wq, bq, Wk, bk, Wv, bv)
    scores = _sc_scores(q, k, row, col)
    cb = _softmax_c(scores)
    acc = _sc_scatter(v, scores,
                      row.reshape(_E // _C2, _C2), col.reshape(_E // _C2, _C2),
                      cb[:, :16])
    return _final(acc[0], acc[1], Wo, bo.reshape(1, _HID))
